# Initial kernel scaffold; baseline (speedup 1.0000x reference)
import jax
import jax.numpy as jnp
from jax.experimental import pallas as pl


def kernel(x, edge_index, W1l, b1, W1r, W2l, b2, W2r, W3l, b3, W3r, Wfc, bfc):
    raise NotImplementedError("write your pallas kernel here")


# speedup vs baseline: 3.0569x; 3.0569x over previous
import functools

import jax
import jax.numpy as jnp
from jax import lax
from jax.experimental import pallas as pl
from jax.experimental.pallas import tpu as pltpu
from jax.experimental.pallas import tpu_sc as plsc

N = 10000
E = 320000
F_IN = 128
H = 256

NC = 2
NS = 16
L = 16

CHUNK = 128
NCH = 158
E_PAD = NS * NCH * CHUNK
NCH2 = NCH // 2
NT = 10240
COV = NT // 2
RACC = COV + 128
RZ = RACC // NS
OPT = COV // NS
RPTD = NT // NS
EW = E // (NC * NS)

_mesh = plsc.VectorSubcoreMesh(core_axis_name="c", subcore_axis_name="s")


@functools.partial(
    pl.kernel,
    out_type=[
        jax.ShapeDtypeStruct((NT, 128), jnp.float32),
        jax.ShapeDtypeStruct((NT, 128), jnp.float32),
    ],
    mesh=_mesh,
    scratch_types=[
        pltpu.VMEM((NCH, CHUNK), jnp.int32),
        pltpu.VMEM((NCH, CHUNK), jnp.int32),
        pltpu.VMEM((NCH, CHUNK), jnp.int32),
        pltpu.VMEM((CHUNK, 128), jnp.float32),
        pltpu.VMEM((16,), jnp.int32),
        pltpu.VMEM_SHARED((RACC, 128), jnp.float32),
    ],
)
def _seg(tab_a, tab_b, srcg, dstg, zrows, params,
         out_a, out_b, sidx, didx, didx2, buf, pv, acc):
    c = lax.axis_index("c")
    s = lax.axis_index("s")
    w = c * NS + s
    pltpu.sync_copy(srcg.at[w], sidx)
    pltpu.sync_copy(dstg.at[w], didx)
    pltpu.sync_copy(params, pv)
    nch = pv[pl.ds(0, L)][0]

    def run(tab, out):
        for lo in (0, COV):
            def rbody(k, carry):
                j = k // (CHUNK // L)
                o = (k % (CHUNK // L)) * L
                d = didx[j, pl.ds(o, L)]
                ok = (d >= lo) & (d < lo + COV)
                didx2[j, pl.ds(o, L)] = jnp.where(ok, d - lo, COV)
                return carry

            lax.fori_loop(0, nch * (CHUNK // L), rbody, 0)
            pltpu.sync_copy(zrows, acc.at[pl.ds(s * RZ, RZ)])
            plsc.subcore_barrier()

            def sbody(j, carry):
                pltpu.sync_copy(tab.at[sidx.at[j]], buf)
                pltpu.sync_copy(buf, acc.at[didx2.at[j]], add=True)
                return carry

            lax.fori_loop(0, nch, sbody, 0)
            plsc.subcore_barrier()
            pltpu.sync_copy(acc.at[pl.ds(s * OPT, OPT)],
                            out.at[pl.ds(lo + s * OPT, OPT)])
            plsc.subcore_barrier()

    @pl.when(c == 0)
    def _():
        run(tab_a, out_a)

    @pl.when(c == 1)
    def _():
        run(tab_b, out_b)


EPW = E_PAD // (NC * NS)
DSL = NT // NS


@functools.partial(
    pl.kernel,
    out_type=jax.ShapeDtypeStruct((NC, NT), jnp.float32),
    mesh=_mesh,
    scratch_types=[
        pltpu.VMEM((EPW,), jnp.int32),
        pltpu.VMEM((NT,), jnp.float32),
        pltpu.VMEM((NS, DSL), jnp.float32),
        pltpu.VMEM((DSL,), jnp.float32),
        pltpu.VMEM_SHARED((NS, NT), jnp.float32),
    ],
    compiler_params=pltpu.CompilerParams(needs_layout_passes=False),
)
def _deg(dstg, out, didx, hist, tmp, res, slab):
    c = lax.axis_index("c")
    s = lax.axis_index("s")
    wid = c * NS + s
    pltpu.sync_copy(dstg.at[wid], didx)
    zero16 = jnp.zeros((L,), jnp.float32)
    one16 = jnp.ones((L,), jnp.float32)

    def zbody(k, carry):
        hist[pl.ds(k * L, L)] = zero16
        return carry

    lax.fori_loop(0, NT // L, zbody, 0)

    def hbody(k, carry):
        d = didx[pl.ds(k * L, L)]
        plsc.addupdate_scatter(hist, [d], one16)
        return carry

    lax.fori_loop(0, EPW // L, hbody, 0)
    pltpu.sync_copy(hist, slab.at[s])
    plsc.subcore_barrier()
    pltpu.sync_copy(slab.at[:, pl.ds(s * DSL, DSL)], tmp)

    def rbody(k, carry):
        acc16 = zero16
        for r in range(NS):
            acc16 = acc16 + tmp[r, pl.ds(k * L, L)]
        res[pl.ds(k * L, L)] = acc16
        return carry

    lax.fori_loop(0, DSL // L, rbody, 0)
    pltpu.sync_copy(res, out.at[c, pl.ds(s * DSL, DSL)])


@functools.partial(
    pl.kernel,
    out_type=jax.ShapeDtypeStruct((E,), jnp.float32),
    mesh=_mesh,
    scratch_types=[
        pltpu.VMEM((EW,), jnp.int32),
        pltpu.VMEM((EW,), jnp.int32),
        pltpu.VMEM((NT,), jnp.float32),
        pltpu.VMEM((NT,), jnp.float32),
        pltpu.VMEM((EW,), jnp.float32),
    ],
    compiler_params=pltpu.CompilerParams(needs_layout_passes=False),
)
def _rate(srcg, dstg, s_h, t_h, out, sidx, didx, sv, tv, obuf):
    c = lax.axis_index("c")
    s = lax.axis_index("s")
    wid = c * NS + s
    pltpu.sync_copy(srcg.at[wid], sidx)
    pltpu.sync_copy(dstg.at[wid], didx)
    pltpu.sync_copy(s_h, sv)
    pltpu.sync_copy(t_h, tv)

    def body(i, carry):
        si = sidx[pl.ds(i * L, L)]
        di = didx[pl.ds(i * L, L)]
        a = plsc.load_gather(sv, [si])
        b = plsc.load_gather(tv, [di])
        r = 4.0 / (1.0 + jnp.exp(-(a + b))) + 1.0
        obuf[pl.ds(i * L, L)] = r
        return carry

    lax.fori_loop(0, EW // L, body, 0)
    pltpu.sync_copy(obuf, out.at[pl.ds(wid * EW, EW)])


_BN = 1280

_row = lambda bd: pl.BlockSpec((_BN, bd), lambda i: (i, 0))
_full = lambda a, b: pl.BlockSpec((a, b), lambda i: (0, 0))


def _make_tc_layer(partial_sums):
    def body(aa, ab, da, db, ha, hb, wl, wr, bb, oa, ob):
        deg = da[...] + db[...]
        inv = 1.0 / jnp.maximum(deg, 1.0)
        if partial_sums:
            agg = (aa[...] + ab[...]) * inv
            h = ha[...]
        else:
            agg = jnp.concatenate([aa[...], ab[...]], axis=1) * inv
            h = jnp.concatenate([ha[...], hb[...]], axis=1)
        z = (
            jnp.dot(agg, wl[...], preferred_element_type=jnp.float32)
            + bb[...]
            + jnp.dot(h, wr[...], preferred_element_type=jnp.float32)
        )
        o = jnp.maximum(z, 0.0)
        oa[...] = o[:, : H // 2]
        ob[...] = o[:, H // 2 :]

    F = F_IN if partial_sums else H
    return pl.pallas_call(
        body,
        grid=(NT // _BN,),
        in_specs=[
            _row(128), _row(128), _row(1), _row(1), _row(128), _row(128),
            _full(F, H), _full(F, H), _full(1, H),
        ],
        out_specs=[_row(H // 2), _row(H // 2)],
        out_shape=[
            jax.ShapeDtypeStruct((NT, H // 2), jnp.float32),
            jax.ShapeDtypeStruct((NT, H // 2), jnp.float32),
        ],
    )


_tc1 = _make_tc_layer(True)
_tc2 = _make_tc_layer(False)


def _tc3_body(aa, ab, da, db, ha, hb, wl, wr, bb, ws, wt, bf, os_, ot):
    deg = da[...] + db[...]
    inv = 1.0 / jnp.maximum(deg, 1.0)
    agg = jnp.concatenate([aa[...], ab[...]], axis=1) * inv
    h = jnp.concatenate([ha[...], hb[...]], axis=1)
    z = (
        jnp.dot(agg, wl[...], preferred_element_type=jnp.float32)
        + bb[...]
        + jnp.dot(h, wr[...], preferred_element_type=jnp.float32)
    )
    o = jnp.maximum(z, 0.0)
    os_[...] = jnp.sum(o * ws[...], axis=1, keepdims=True) + bf[0, 0]
    ot[...] = jnp.sum(o * wt[...], axis=1, keepdims=True)


_tc3 = pl.pallas_call(
    _tc3_body,
    grid=(NT // _BN,),
    in_specs=[
        _row(128), _row(128), _row(1), _row(1), _row(128), _row(128),
        _full(H, H), _full(H, H), _full(1, H), _full(1, H), _full(1, H),
        _full(1, 1),
    ],
    out_specs=[_row(1), _row(1)],
    out_shape=[
        jax.ShapeDtypeStruct((NT, 1), jnp.float32),
        jax.ShapeDtypeStruct((NT, 1), jnp.float32),
    ],
)


def kernel(x, edge_index, W1l, b1, W1r, W2l, b2, W2r, W3l, b3, W3r, Wfc, bfc):
    src = edge_index[0]
    dst = edge_index[1]
    pad = E_PAD - E
    srcp = jnp.concatenate([src, jnp.full((pad,), NT - 1, jnp.int32)])
    dstp = jnp.concatenate([dst, jnp.full((pad,), N, jnp.int32)])
    s16 = srcp.reshape(NS, NCH, CHUNK)
    d16 = dstp.reshape(NS, NCH, CHUNK)
    src_f = jnp.concatenate([s16, s16], axis=0)
    dst_f = jnp.concatenate([d16, d16], axis=0)
    s32 = srcp.reshape(NC * NS, NCH2, CHUNK)
    d32 = dstp.reshape(NC * NS, NCH2, CHUNK)
    src_e = jnp.pad(s32, ((0, 0), (0, NCH - NCH2), (0, 0)))
    dst_e = jnp.pad(d32, ((0, 0), (0, NCH - NCH2), (0, 0)))
    src_r = src.reshape(NC * NS, EW)
    dst_r = dst.reshape(NC * NS, EW)

    dst_w = dstp.reshape(NC * NS, EPW)

    z128 = jnp.zeros((RZ, 128), jnp.float32)
    p_e = jnp.full((16,), NCH2, jnp.int32)
    p_f = jnp.full((16,), NCH, jnp.int32)

    xp = jnp.pad(x, ((0, NT - N), (0, 0)))

    dg = _deg(dst_w)
    deg_a = dg[0].reshape(NT, 1)
    deg_b = dg[1].reshape(NT, 1)

    agg1a, agg1b = _seg(xp, xp, src_e, dst_e, z128, p_e)
    h1a, h1b = _tc1(agg1a, agg1b, deg_a, deg_b, xp, xp,
                    W1l.T, W1r.T, b1.reshape(1, H))

    agg2a, agg2b = _seg(h1a, h1b, src_f, dst_f, z128, p_f)
    h2a, h2b = _tc2(agg2a, agg2b, deg_a, deg_b, h1a, h1b,
                    W2l.T, W2r.T, b2.reshape(1, H))

    agg3a, agg3b = _seg(h2a, h2b, src_f, dst_f, z128, p_f)
    sv, tv = _tc3(agg3a, agg3b, deg_a, deg_b, h2a, h2b,
                  W3l.T, W3r.T, b3.reshape(1, H),
                  Wfc[:, :H], Wfc[:, H:], bfc.reshape(1, 1))

    return _rate(src_r, dst_r, sv.reshape(NT), tv.reshape(NT))
